# baseline (speedup 1.0000x reference)
import functools

import jax
import jax.numpy as jnp
from jax import lax
from jax.experimental import pallas as pl
from jax.experimental.pallas import tpu as pltpu
from jax.experimental.pallas import tpu_sc as plsc

_TOTAL = 128 * 128
_ROWS = 128
_D = 4096
_NC, _NS = 2, 16
_NW = _NC * _NS
_B_PER_W = _TOTAL // _NW
_G = 16
_NSLOT = 8
_N_GROUPS = _B_PER_W // _G

_mesh = plsc.VectorSubcoreMesh(core_axis_name="c", subcore_axis_name="s")


@functools.partial(
    pl.kernel,
    out_type=jax.ShapeDtypeStruct((_TOTAL, _D), jnp.float32),
    mesh=_mesh,
    scratch_types=[
        pltpu.VMEM((_B_PER_W,), jnp.int32),
        pltpu.VMEM_SHARED((_ROWS, _D), jnp.float32),
        pltpu.SemaphoreType.DMA((_NSLOT,)),
    ],
)
def _gather_kernel(idx_hbm, table_hbm, out_hbm, idx_v, table_sp, sems):
    sid = lax.axis_index("s")
    wid = sid * _NC + lax.axis_index("c")
    base = wid * _B_PER_W

    rows_per_sub = _ROWS // _NS
    pltpu.sync_copy(
        table_hbm.at[pl.ds(sid * rows_per_sub, rows_per_sub)],
        table_sp.at[pl.ds(sid * rows_per_sub, rows_per_sub)],
    )
    pltpu.sync_copy(idx_hbm.at[pl.ds(base, _B_PER_W)], idx_v)
    plsc.subcore_barrier()

    def start_group(g, s):
        voff = pl.multiple_of(g * _G, 16)
        vec = idx_v[pl.ds(voff, 16)]
        for j in range(_G):
            r = vec[j]
            pltpu.async_copy(
                table_sp.at[pl.ds(r, 1)],
                out_hbm.at[pl.ds(base + g * _G + j, 1)],
                sems.at[s],
            )

    def wait_group(s):
        pltpu.make_async_copy(
            table_sp.at[pl.ds(0, _G)],
            out_hbm.at[pl.ds(base, _G)],
            sems.at[s],
        ).wait()

    for s in range(_NSLOT):
        start_group(s, s)

    def outer(o, _):
        for s in range(_NSLOT):
            g = o * _NSLOT + s
            wait_group(s)

            @pl.when(g + _NSLOT < _N_GROUPS)
            def _():
                start_group(g + _NSLOT, s)

        return ()

    lax.fori_loop(0, _N_GROUPS // _NSLOT, outer, (), unroll=False)


def kernel(indices, embedding_weight):
    flat_idx = indices.reshape(-1).astype(jnp.int32)
    out = _gather_kernel(flat_idx, embedding_weight)
    return out.reshape(indices.shape[0], indices.shape[1], _D)


# speedup vs baseline: 1.0172x; 1.0172x over previous
import functools

import jax
import jax.numpy as jnp
from jax import lax
from jax.experimental import pallas as pl
from jax.experimental.pallas import tpu as pltpu
from jax.experimental.pallas import tpu_sc as plsc

_TOTAL = 128 * 128
_ROWS = 128
_D = 4096
_NC, _NS = 2, 16
_NW = _NC * _NS
_B_PER_W = _TOTAL // _NW
_G = 32
_NSLOT = 2
_N_GROUPS = _B_PER_W // _G

_mesh = plsc.VectorSubcoreMesh(core_axis_name="c", subcore_axis_name="s")


@functools.partial(
    pl.kernel,
    out_type=jax.ShapeDtypeStruct((_TOTAL, _D), jnp.float32),
    mesh=_mesh,
    scratch_types=[
        pltpu.VMEM((_B_PER_W,), jnp.int32),
        pltpu.VMEM_SHARED((_ROWS, _D), jnp.float32),
        pltpu.SemaphoreType.DMA((_NSLOT,)),
    ],
)
def _gather_kernel(idx_hbm, table_hbm, out_hbm, idx_v, table_sp, sems):
    sid = lax.axis_index("s")
    wid = sid * _NC + lax.axis_index("c")
    base = wid * _B_PER_W

    pltpu.sync_copy(idx_hbm.at[pl.ds(base, _B_PER_W)], idx_v)

    rows_per_sub = _ROWS // _NS
    pltpu.sync_copy(
        table_hbm.at[pl.ds(sid * rows_per_sub, rows_per_sub)],
        table_sp.at[pl.ds(sid * rows_per_sub, rows_per_sub)],
    )
    plsc.subcore_barrier()

    def start_group(g, s):
        for w in range(_G // 16):
            voff = pl.multiple_of(g * _G + w * 16, 16)
            vec = idx_v[pl.ds(voff, 16)]
            for j in range(16):
                r = vec[j]
                pltpu.async_copy(
                    table_sp.at[pl.ds(r, 1)],
                    out_hbm.at[pl.ds(base + g * _G + w * 16 + j, 1)],
                    sems.at[s],
                )

    def wait_group(s):
        pltpu.make_async_copy(
            table_sp.at[pl.ds(0, _G)],
            out_hbm.at[pl.ds(base, _G)],
            sems.at[s],
        ).wait()

    for s in range(_NSLOT):
        start_group(s, s)

    def outer(o, _):
        for s in range(_NSLOT):
            g = o * _NSLOT + s
            wait_group(s)

            @pl.when(g + _NSLOT < _N_GROUPS)
            def _():
                start_group(g + _NSLOT, s)

        return ()

    lax.fori_loop(0, _N_GROUPS // _NSLOT, outer, (), unroll=False)

    for s in range(_N_GROUPS % _NSLOT):
        wait_group(s)


def kernel(indices, embedding_weight):
    flat_idx = indices.reshape(-1).astype(jnp.int32)
    out = _gather_kernel(flat_idx, embedding_weight)
    return out.reshape(indices.shape[0], indices.shape[1], _D)


# speedup vs baseline: 1.0408x; 1.0233x over previous
import functools

import jax
import jax.numpy as jnp
from jax import lax
from jax.experimental import pallas as pl
from jax.experimental.pallas import tpu as pltpu
from jax.experimental.pallas import tpu_sc as plsc

_TOTAL = 128 * 128
_ROWS = 128
_D = 4096
_NC, _NS = 2, 16
_NW = _NC * _NS
_B_PER_W = _TOTAL // _NW

_G = 16
_NSLOT = 3
_A_ROWS = 384
_NA = _A_ROWS // _G

_CHUNK = 8
_NBUF = 2
_B_ROWS = _B_PER_W - _A_ROWS
_NB = _B_ROWS // _CHUNK

_N_OUTER = 8

_mesh = plsc.VectorSubcoreMesh(core_axis_name="c", subcore_axis_name="s")


@functools.partial(
    pl.kernel,
    out_type=jax.ShapeDtypeStruct((_TOTAL, _D), jnp.float32),
    mesh=_mesh,
    scratch_types=[
        pltpu.VMEM((_B_PER_W,), jnp.int32),
        pltpu.VMEM((_NBUF, _CHUNK, _D), jnp.float32),
        pltpu.VMEM_SHARED((_ROWS, _D), jnp.float32),
        pltpu.SemaphoreType.DMA((_NSLOT,)),
        pltpu.SemaphoreType.DMA((_NBUF,)),
        pltpu.SemaphoreType.DMA((_NBUF,)),
    ],
)
def _gather_kernel(
    idx_hbm, table_hbm, out_hbm, idx_v, bufs, table_sp, asems, gsems, wsems
):
    sid = lax.axis_index("s")
    wid = sid * _NC + lax.axis_index("c")
    base = wid * _B_PER_W

    pltpu.sync_copy(idx_hbm.at[pl.ds(base, _B_PER_W)], idx_v)

    rows_per_sub = _ROWS // _NS
    pltpu.sync_copy(
        table_hbm.at[pl.ds(sid * rows_per_sub, rows_per_sub)],
        table_sp.at[pl.ds(sid * rows_per_sub, rows_per_sub)],
    )
    plsc.subcore_barrier()

    def start_group(g, s):
        voff = pl.multiple_of(g * _G, 16)
        vec = idx_v[pl.ds(voff, 16)]
        for j in range(_G):
            r = vec[j]
            pltpu.async_copy(
                table_sp.at[pl.ds(r, 1)],
                out_hbm.at[pl.ds(base + g * _G + j, 1)],
                asems.at[s],
            )

    def wait_group(s):
        pltpu.make_async_copy(
            table_sp.at[pl.ds(0, _G)],
            out_hbm.at[pl.ds(base, _G)],
            asems.at[s],
        ).wait()

    def start_gather(c, b):
        pltpu.async_copy(
            table_hbm.at[idx_v.at[pl.ds(_A_ROWS + c * _CHUNK, _CHUNK)]],
            bufs.at[b],
            gsems.at[b],
        )

    def wait_gather(b):
        pltpu.make_async_copy(
            table_hbm.at[pl.ds(0, _CHUNK)], bufs.at[b], gsems.at[b]
        ).wait()

    def start_write(c, b):
        pltpu.async_copy(
            bufs.at[b],
            out_hbm.at[pl.ds(base + _A_ROWS + c * _CHUNK, _CHUNK)],
            wsems.at[b],
        )

    def wait_write(b):
        pltpu.make_async_copy(
            bufs.at[b], out_hbm.at[pl.ds(base, _CHUNK)], wsems.at[b]
        ).wait()

    for b in range(_NBUF):
        start_gather(b, b)
    for s in range(_NSLOT):
        start_group(s, s)

    def outer(o, _):
        for k in range(_NSLOT):
            g = o * _NSLOT + k
            wait_group(k)

            @pl.when(g + _NSLOT < _NA)
            def _():
                start_group(g + _NSLOT, k)

        for b in range(_NBUF):
            c = o * _NBUF + b
            wait_gather(b)
            start_write(c, b)

            @pl.when(c + _NBUF < _NB)
            def _():
                wait_write(b)
                start_gather(c + _NBUF, b)

        return ()

    lax.fori_loop(0, _N_OUTER, outer, (), unroll=False)

    for b in range(_NBUF):
        wait_write(b)


def kernel(indices, embedding_weight):
    flat_idx = indices.reshape(-1).astype(jnp.int32)
    out = _gather_kernel(flat_idx, embedding_weight)
    return out.reshape(indices.shape[0], indices.shape[1], _D)


# speedup vs baseline: 1.0425x; 1.0016x over previous
import functools

import jax
import jax.numpy as jnp
from jax import lax
from jax.experimental import pallas as pl
from jax.experimental.pallas import tpu as pltpu
from jax.experimental.pallas import tpu_sc as plsc

_TOTAL = 128 * 128
_ROWS = 128
_D = 4096
_NC, _NS = 2, 16
_NW = _NC * _NS
_B_PER_W = _TOTAL // _NW

_G = 16
_NSLOT = 3
_A_ROWS = 384
_NA = _A_ROWS // _G

_CHUNK = 8
_NBUF = 2
_B_ROWS = _B_PER_W - _A_ROWS
_NB = _B_ROWS // _CHUNK

_N_OUTER = _NA // _NSLOT

_mesh = plsc.VectorSubcoreMesh(core_axis_name="c", subcore_axis_name="s")


@functools.partial(
    pl.kernel,
    out_type=jax.ShapeDtypeStruct((_TOTAL, _D), jnp.float32),
    mesh=_mesh,
    scratch_types=[
        pltpu.VMEM((_B_PER_W,), jnp.int32),
        pltpu.VMEM((_NBUF, _CHUNK, _D), jnp.float32),
        pltpu.VMEM_SHARED((_ROWS, _D), jnp.float32),
        pltpu.SemaphoreType.DMA((_NSLOT,)),
        pltpu.SemaphoreType.DMA((_NBUF,)),
        pltpu.SemaphoreType.DMA((_NBUF,)),
    ],
)
def _gather_kernel(
    idx_hbm, table_hbm, out_hbm, idx_v, bufs, table_sp, asems, gsems, wsems
):
    sid = lax.axis_index("s")
    wid = sid * _NC + lax.axis_index("c")
    base = wid * _B_PER_W

    pltpu.sync_copy(idx_hbm.at[pl.ds(base, _B_PER_W)], idx_v)

    rows_per_sub = _ROWS // _NS
    pltpu.sync_copy(
        table_hbm.at[pl.ds(sid * rows_per_sub, rows_per_sub)],
        table_sp.at[pl.ds(sid * rows_per_sub, rows_per_sub)],
    )
    plsc.subcore_barrier()

    def start_group(g, s):
        voff = pl.multiple_of(g * _G, 16)
        vec = idx_v[pl.ds(voff, 16)]
        for j in range(_G):
            r = vec[j]
            pltpu.async_copy(
                table_sp.at[pl.ds(r, 1)],
                out_hbm.at[pl.ds(base + g * _G + j, 1)],
                asems.at[s],
            )

    def wait_group(s):
        pltpu.make_async_copy(
            table_sp.at[pl.ds(0, _G)],
            out_hbm.at[pl.ds(base, _G)],
            asems.at[s],
        ).wait()

    def start_gather(c, b):
        pltpu.async_copy(
            table_hbm.at[idx_v.at[pl.ds(_A_ROWS + c * _CHUNK, _CHUNK)]],
            bufs.at[b],
            gsems.at[b],
        )

    def wait_gather(b):
        pltpu.make_async_copy(
            table_hbm.at[pl.ds(0, _CHUNK)], bufs.at[b], gsems.at[b]
        ).wait()

    def start_write(c, b):
        pltpu.async_copy(
            bufs.at[b],
            out_hbm.at[pl.ds(base + _A_ROWS + c * _CHUNK, _CHUNK)],
            wsems.at[b],
        )

    def wait_write(b):
        pltpu.make_async_copy(
            bufs.at[b], out_hbm.at[pl.ds(base, _CHUNK)], wsems.at[b]
        ).wait()

    for b in range(_NBUF):
        start_gather(b, b)
    for s in range(_NSLOT):
        start_group(s, s)

    def outer(o, _):
        for k in range(_NSLOT):
            g = o * _NSLOT + k
            wait_group(k)

            @pl.when(g + _NSLOT < _NA)
            def _():
                start_group(g + _NSLOT, k)

        for b in range(_NBUF):
            c = o * _NBUF + b
            wait_gather(b)
            start_write(c, b)

            @pl.when(c + _NBUF < _NB)
            def _():
                wait_write(b)
                start_gather(c + _NBUF, b)

        return ()

    lax.fori_loop(0, _N_OUTER, outer, (), unroll=False)

    for b in range(_NBUF):
        wait_write(b)


def kernel(indices, embedding_weight):
    flat_idx = indices.reshape(-1).astype(jnp.int32)
    out = _gather_kernel(flat_idx, embedding_weight)
    return out.reshape(indices.shape[0], indices.shape[1], _D)
